# baseline (speedup 1.0000x reference)
import functools

import jax
import jax.numpy as jnp
from jax import lax
from jax.experimental import pallas as pl
from jax.experimental.pallas import tpu as pltpu
from jax.experimental.pallas import tpu_sc as plsc

NWORDS_K = 1_000_000
EMB_K = 64
SEQ_K = 200

_NC = 2
_NS = 16
_NW = _NC * _NS
_B_PAD = 256
_BPW = _B_PAD // _NW

_V_TILE = 16384
_GRID = (NWORDS_K + _V_TILE - 1) // _V_TILE


def _sc_gather(table, idx):
    mesh = plsc.VectorSubcoreMesh(core_axis_name="c", subcore_axis_name="s")

    @functools.partial(
        pl.kernel,
        mesh=mesh,
        out_type=jax.ShapeDtypeStruct((_B_PAD, EMB_K), jnp.float32),
        scratch_types=[
            pltpu.VMEM((_BPW,), jnp.int32),
            pltpu.VMEM((_BPW, EMB_K), jnp.float32),
            pltpu.SemaphoreType.DMA,
        ],
        compiler_params=pltpu.CompilerParams(use_tc_tiling_on_sc=False),
    )
    def k(table_hbm, idx_hbm, out_hbm, idx_v, rows_v, sem):
        wid = lax.axis_index("s") * _NC + lax.axis_index("c")
        base = wid * _BPW
        pltpu.sync_copy(idx_hbm.at[pl.ds(base, _BPW)], idx_v)
        pltpu.async_copy(table_hbm.at[idx_v], rows_v, sem).wait()
        pltpu.sync_copy(rows_v, out_hbm.at[pl.ds(base, _BPW)])

    return k(table, idx)


def _tc_body(rows_ref, w_ref, b_ref, out_ref):
    rows = rows_ref[...]
    mask = (lax.broadcasted_iota(jnp.int32, (_B_PAD, 1), 0) < SEQ_K)
    s = jnp.sum(jnp.where(mask, rows, 0.0), axis=0, keepdims=True)
    acc = lax.dot_general(
        s, w_ref[...], (((1,), (1,)), ((), ())),
        preferred_element_type=jnp.float32,
    )
    out_ref[...] = acc + b_ref[...]


def _tc_matvec(rows, proj_W, b2):
    return pl.pallas_call(
        _tc_body,
        grid=(_GRID,),
        in_specs=[
            pl.BlockSpec((_B_PAD, EMB_K), lambda i: (0, 0)),
            pl.BlockSpec((_V_TILE, EMB_K), lambda i: (i, 0)),
            pl.BlockSpec((1, _V_TILE), lambda i: (0, i)),
        ],
        out_specs=pl.BlockSpec((1, _V_TILE), lambda i: (0, i)),
        out_shape=jax.ShapeDtypeStruct((1, NWORDS_K), jnp.float32),
    )(rows, proj_W, b2)


def kernel(words, emb_table, proj_W, proj_b):
    idx = jnp.zeros((_B_PAD,), jnp.int32).at[:SEQ_K].set(words.astype(jnp.int32))
    rows = _sc_gather(emb_table, idx)
    b2 = proj_b.reshape(1, NWORDS_K)
    return _tc_matvec(rows, proj_W, b2)


# speedup vs baseline: 1.2933x; 1.2933x over previous
import functools

import jax
import jax.numpy as jnp
from jax import lax
from jax.experimental import pallas as pl
from jax.experimental.pallas import tpu as pltpu
from jax.experimental.pallas import tpu_sc as plsc

NWORDS_K = 1_000_000
EMB_K = 64
SEQ_K = 200

_NC = 2
_NS = 16
_NW = _NC * _NS
_B_PAD = 256
_BPW = _B_PAD // _NW

_V_TILE = 16384
_GRID = (NWORDS_K + _V_TILE - 1) // _V_TILE


_PER_SCS = _B_PAD // _NC


def _sc_gather_rows(table, idx):
    mesh = plsc.ScalarSubcoreMesh(axis_name="c", num_cores=_NC)

    @functools.partial(
        pl.kernel,
        mesh=mesh,
        out_type=jax.ShapeDtypeStruct((_B_PAD, EMB_K), jnp.float32),
        scratch_types=[
            pltpu.SMEM((_PER_SCS,), jnp.int32),
            pltpu.SemaphoreType.DMA,
        ],
    )
    def k(table_hbm, idx_hbm, out_hbm, idx_s, sem):
        cid = lax.axis_index("c")
        base = cid * _PER_SCS
        pltpu.sync_copy(idx_hbm.at[pl.ds(base, _PER_SCS)], idx_s)
        copies = []
        for j in range(_PER_SCS):
            copies.append(pltpu.async_copy(
                table_hbm.at[pl.ds(idx_s[j], 1)],
                out_hbm.at[pl.ds(base + j, 1)],
                sem,
            ))
        for c in copies:
            c.wait()

    return k(table, idx)


def _tc_body(rows_ref, wt_ref, w_ref, b_ref, out_ref):
    s = lax.dot_general(
        wt_ref[...], rows_ref[...], (((1,), (0,)), ((), ())),
        preferred_element_type=jnp.float32,
    )
    acc = lax.dot_general(
        s, w_ref[...], (((1,), (1,)), ((), ())),
        preferred_element_type=jnp.float32,
    )
    out_ref[...] = acc + b_ref[...]


def _tc_matvec(rows, weights, proj_W, b2):
    return pl.pallas_call(
        _tc_body,
        grid=(_GRID,),
        in_specs=[
            pl.BlockSpec((_B_PAD, EMB_K), lambda i: (0, 0)),
            pl.BlockSpec((1, _B_PAD), lambda i: (0, 0)),
            pl.BlockSpec((_V_TILE, EMB_K), lambda i: (i, 0)),
            pl.BlockSpec((1, _V_TILE), lambda i: (0, i)),
        ],
        out_specs=pl.BlockSpec((1, _V_TILE), lambda i: (0, i)),
        out_shape=jax.ShapeDtypeStruct((1, NWORDS_K), jnp.float32),
    )(rows, weights, proj_W, b2)


def kernel(words, emb_table, proj_W, proj_b):
    w32 = words.astype(jnp.int32)
    idx = jnp.zeros((_B_PAD,), jnp.int32).at[:SEQ_K].set(w32)
    weights = (jnp.arange(_B_PAD, dtype=jnp.int32) < SEQ_K).astype(jnp.float32)
    rows = _sc_gather_rows(emb_table, idx)
    b2 = proj_b.reshape(1, NWORDS_K)
    return _tc_matvec(rows, weights.reshape(1, _B_PAD), proj_W, b2)


# speedup vs baseline: 3.0949x; 2.3930x over previous
import functools

import jax
import jax.numpy as jnp
from jax import lax
from jax.experimental import pallas as pl
from jax.experimental.pallas import tpu as pltpu
from jax.experimental.pallas import tpu_sc as plsc

NWORDS_K = 1_000_000
EMB_K = 64
SEQ_K = 200

_NC = 2
_B_PAD = 256
_PER_SCS = _B_PAD // _NC
_LANES = 128
_CW = _B_PAD * _LANES

_V_TILE = 16384
_GRID = (NWORDS_K + _V_TILE - 1) // _V_TILE


def _sc_gather_cols(table_t, idx):
    mesh = plsc.ScalarSubcoreMesh(axis_name="c", num_cores=_NC)

    @functools.partial(
        pl.kernel,
        mesh=mesh,
        out_type=jax.ShapeDtypeStruct((EMB_K, _CW), jnp.float32),
        scratch_types=[
            pltpu.SMEM((_PER_SCS,), jnp.int32),
            pltpu.SemaphoreType.DMA,
        ],
    )
    def k(table_hbm, idx_hbm, out_hbm, idx_s, sem):
        cid = lax.axis_index("c")
        base = cid * _PER_SCS
        pltpu.sync_copy(idx_hbm.at[pl.ds(base, _PER_SCS)], idx_s)
        copies = []
        for j in range(_PER_SCS):
            off = pl.multiple_of(idx_s[j], _LANES)
            copies.append(pltpu.async_copy(
                table_hbm.at[:, pl.ds(off, _LANES)],
                out_hbm.at[:, pl.ds((base + j) * _LANES, _LANES)],
                sem,
            ))
        for c in copies:
            c.wait()

    return k(table_t, idx)


def _tc_body(u_ref, cols_ref, wt_ref, b_ref, out_ref, s_ref):
    @pl.when(pl.program_id(0) == 0)
    def _():
        s_ref[...] = lax.dot_general(
            u_ref[...], cols_ref[...], (((1,), (1,)), ((), ())),
            preferred_element_type=jnp.float32,
        )

    acc = lax.dot_general(
        s_ref[...], wt_ref[...], (((1,), (0,)), ((), ())),
        preferred_element_type=jnp.float32,
    )
    out_ref[...] = acc + b_ref[...]


def _tc_matvec(u, cols, w_t, b2):
    return pl.pallas_call(
        _tc_body,
        grid=(_GRID,),
        in_specs=[
            pl.BlockSpec((1, _CW), lambda i: (0, 0)),
            pl.BlockSpec((EMB_K, _CW), lambda i: (0, 0)),
            pl.BlockSpec((EMB_K, _V_TILE), lambda i: (0, i)),
            pl.BlockSpec((1, _V_TILE), lambda i: (0, i)),
        ],
        out_specs=pl.BlockSpec((1, _V_TILE), lambda i: (0, i)),
        out_shape=jax.ShapeDtypeStruct((1, NWORDS_K), jnp.float32),
        scratch_shapes=[pltpu.VMEM((1, EMB_K), jnp.float32)],
    )(u, cols, w_t, b2)


def kernel(words, emb_table, proj_W, proj_b):
    w32 = words.astype(jnp.int32)
    idx = jnp.zeros((_B_PAD,), jnp.int32).at[:SEQ_K].set(
        (w32 // _LANES) * _LANES)
    pos = jnp.arange(SEQ_K, dtype=jnp.int32) * _LANES + (w32 % _LANES)
    u = jnp.zeros((_CW,), jnp.float32).at[pos].add(1.0)
    emb_t = emb_table.T
    w_t = proj_W.T
    cols = _sc_gather_cols(emb_t, idx)
    b2 = proj_b.reshape(1, NWORDS_K)
    return _tc_matvec(u.reshape(1, _CW), cols, w_t, b2)
